# Initial kernel scaffold; baseline (speedup 1.0000x reference)
import jax
import jax.numpy as jnp
from jax.experimental import pallas as pl


def kernel(distances, prototype_labels):
    raise NotImplementedError("write your pallas kernel here")


# speedup vs baseline: 1.9480x; 1.9480x over previous
import functools

import jax
import jax.numpy as jnp
from jax import lax
from jax.experimental import pallas as pl
from jax.experimental.pallas import tpu as pltpu
from jax.experimental.pallas import tpu_sc as plsc

BATCH = 16384
N_PROTO = 256
N_WORKERS = 32
ROWS_PER_WORKER = BATCH // N_WORKERS
GROUP = 16
N_GROUPS = ROWS_PER_WORKER // GROUP
PITCH = N_PROTO + 1


def _wtac_body(dist_hbm, labels_hbm, out_hbm, labels_v, tile, out_v):
    wid = lax.axis_index("c") * 16 + lax.axis_index("s")
    base = wid * ROWS_PER_WORKER

    pltpu.sync_copy(labels_hbm, labels_v)

    row_iota = lax.iota(jnp.int32, 16)

    def do_group(g, carry):
        tbase = base + g * GROUP
        pltpu.sync_copy(
            dist_hbm.at[pl.ds(tbase, GROUP), :],
            tile.at[:, pl.ds(0, N_PROTO)],
        )

        def do_col(c, st):
            best, bidx = st
            col = jnp.full((16,), c, jnp.int32)
            v = plsc.load_gather(tile, [row_iota, col])
            pred = v < best
            best = jnp.where(pred, v, best)
            bidx = jnp.where(pred, col, bidx)
            return best, bidx

        init = (jnp.full((16,), jnp.inf, jnp.float32),
                jnp.zeros((16,), jnp.int32))
        _, bidx = lax.fori_loop(0, N_PROTO, do_col, init, unroll=8)

        lab = plsc.load_gather(labels_v, [bidx])
        out_v[pl.ds(g * GROUP, GROUP)] = lab
        return carry

    lax.fori_loop(0, N_GROUPS, do_group, 0)
    pltpu.sync_copy(out_v, out_hbm.at[pl.ds(base, ROWS_PER_WORKER)])


@functools.partial(jax.jit, static_argnames=())
def _wtac(distances, labels):
    mesh = plsc.VectorSubcoreMesh(core_axis_name="c", subcore_axis_name="s")
    run = pl.kernel(
        _wtac_body,
        out_type=jax.ShapeDtypeStruct((BATCH,), jnp.int32),
        mesh=mesh,
        scratch_types=[
            pltpu.VMEM((N_PROTO,), jnp.int32),
            pltpu.VMEM((GROUP, PITCH), jnp.float32),
            pltpu.VMEM((ROWS_PER_WORKER,), jnp.int32),
        ],
        compiler_params=pltpu.CompilerParams(
            use_tc_tiling_on_sc=False, needs_layout_passes=False
        ),
        name="wtac_sc",
    )
    return run(distances, labels)


def kernel(distances, prototype_labels):
    labels = prototype_labels.astype(jnp.int32)
    return _wtac(distances, labels)


# speedup vs baseline: 2.3275x; 1.1948x over previous
import functools

import jax
import jax.numpy as jnp
from jax import lax
from jax.experimental import pallas as pl
from jax.experimental.pallas import tpu as pltpu
from jax.experimental.pallas import tpu_sc as plsc

BATCH = 16384
N_PROTO = 256
N_WORKERS = 32
ROWS_PER_WORKER = BATCH // N_WORKERS
GROUP = 16
TILE = 128
N_TILES = ROWS_PER_WORKER // TILE
GROUPS_PER_TILE = TILE // GROUP
PITCH = N_PROTO + 1
K = 8
SPAN = N_PROTO // K


def _wtac_body(dist_hbm, labels_hbm, out_hbm, labels_v, tiles, out_v, sems):
    wid = lax.axis_index("c") * 16 + lax.axis_index("s")
    base = wid * ROWS_PER_WORKER

    pltpu.sync_copy(labels_hbm, labels_v)

    row_iota = lax.iota(jnp.int32, 16)

    def start_tile_copy(t, buf):
        return pltpu.async_copy(
            dist_hbm.at[pl.ds(base + t * TILE, TILE), :],
            tiles[buf].at[:, pl.ds(0, N_PROTO)],
            sems[buf],
        )

    def do_group(tile, t, g):
        r0 = g * GROUP
        rows = jnp.full((16,), r0, jnp.int32) + row_iota

        def do_col(c, st):
            new = []
            for k in range(K):
                best, bidx = st[2 * k], st[2 * k + 1]
                col = jnp.full((16,), c + k * SPAN, jnp.int32)
                v = plsc.load_gather(tile, [rows, col])
                pred = v < best
                new.append(jnp.where(pred, v, best))
                new.append(jnp.where(pred, col, bidx))
            return tuple(new)

        init = []
        for k in range(K):
            init.append(jnp.full((16,), jnp.inf, jnp.float32))
            init.append(jnp.zeros((16,), jnp.int32))
        st = lax.fori_loop(0, SPAN, do_col, tuple(init), unroll=2)

        vals = [st[2 * k] for k in range(K)]
        idxs = [st[2 * k + 1] for k in range(K)]
        n = K
        while n > 1:
            for k in range(n // 2):
                a, b = 2 * k, 2 * k + 1
                pred = vals[b] < vals[a]
                vals[k] = jnp.where(pred, vals[b], vals[a])
                idxs[k] = jnp.where(pred, idxs[b], idxs[a])
            n //= 2

        lab = plsc.load_gather(labels_v, [idxs[0]])
        out_v[pl.ds(t * TILE + r0, GROUP)] = lab

    cp = start_tile_copy(0, 0)
    for t in range(N_TILES):
        nxt = None
        if t + 1 < N_TILES:
            nxt = start_tile_copy(t + 1, (t + 1) % 2)
        cp.wait()
        for g in range(GROUPS_PER_TILE):
            do_group(tiles[t % 2], t, g)
        cp = nxt

    pltpu.sync_copy(out_v, out_hbm.at[pl.ds(base, ROWS_PER_WORKER)])


@jax.jit
def _wtac(distances, labels):
    mesh = plsc.VectorSubcoreMesh(core_axis_name="c", subcore_axis_name="s")
    run = pl.kernel(
        _wtac_body,
        out_type=jax.ShapeDtypeStruct((BATCH,), jnp.int32),
        mesh=mesh,
        scratch_types=[
            pltpu.VMEM((N_PROTO,), jnp.int32),
            [pltpu.VMEM((TILE, PITCH), jnp.float32)] * 2,
            pltpu.VMEM((ROWS_PER_WORKER,), jnp.int32),
            [pltpu.SemaphoreType.DMA] * 2,
        ],
        compiler_params=pltpu.CompilerParams(
            use_tc_tiling_on_sc=False, needs_layout_passes=False
        ),
        name="wtac_sc",
    )
    return run(distances, labels)


def kernel(distances, prototype_labels):
    labels = prototype_labels.astype(jnp.int32)
    return _wtac(distances, labels)


# speedup vs baseline: 2.7588x; 1.1853x over previous
import jax
import jax.numpy as jnp
from jax import lax
from jax.experimental import pallas as pl
from jax.experimental.pallas import tpu as pltpu
from jax.experimental.pallas import tpu_sc as plsc

BATCH = 16384
N_PROTO = 256
N_WORKERS = 32
ROWS_PER_WORKER = BATCH // N_WORKERS
GROUP = 16
N_GROUPS = ROWS_PER_WORKER // GROUP
N_CHUNKS = N_PROTO // 16
BIG = 2**30


def _wtac_body(dist_hbm, labels_hbm, out_hbm, labels_v, tile, out_v):
    wid = lax.axis_index("c") * 16 + lax.axis_index("s")
    base = wid * ROWS_PER_WORKER


    pltpu.sync_copy(labels_hbm, labels_v)

    col_iota = lax.iota(jnp.int32, 16)

    def do_group(g, carry):
        tbase = base + g * GROUP
        pltpu.sync_copy(dist_hbm.at[pl.ds(tbase, GROUP), :], tile)

        labvec = jnp.zeros((16,), jnp.int32)
        for r in range(GROUP):
            best = tile[r, pl.ds(0, 16)]
            bidx = col_iota
            for j in range(1, N_CHUNKS):
                v = tile[r, pl.ds(j * 16, 16)]
                col = col_iota + (j * 16)
                pred = v < best
                best = jnp.where(pred, v, best)
                bidx = jnp.where(pred, col, bidx)
            m = jnp.min(best)
            cand = jnp.where(best == m, bidx, jnp.full((16,), BIG, jnp.int32))
            w = jnp.min(cand)
            labvec = jnp.where(col_iota == r, w, labvec)
        out_v[pl.ds(g * GROUP, GROUP)] = plsc.load_gather(labels_v, [labvec])
        return carry

    lax.fori_loop(0, N_GROUPS, do_group, 0)
    pltpu.sync_copy(out_v, out_hbm.at[pl.ds(base, ROWS_PER_WORKER)])


@jax.jit
def _wtac(distances, labels):
    mesh = plsc.VectorSubcoreMesh(core_axis_name="c", subcore_axis_name="s")
    run = pl.kernel(
        _wtac_body,
        out_type=jax.ShapeDtypeStruct((BATCH,), jnp.int32),
        mesh=mesh,
        scratch_types=[
            pltpu.VMEM((N_PROTO,), jnp.int32),
            pltpu.VMEM((GROUP, N_PROTO), jnp.float32),
            pltpu.VMEM((ROWS_PER_WORKER,), jnp.int32),
        ],
        compiler_params=pltpu.CompilerParams(
            use_tc_tiling_on_sc=True, needs_layout_passes=False
        ),
        name="wtac_sc",
    )
    return run(distances, labels)


def kernel(distances, prototype_labels):
    labels = prototype_labels.astype(jnp.int32)
    return _wtac(distances, labels)


# speedup vs baseline: 4.2360x; 1.5355x over previous
import jax
import jax.numpy as jnp
from jax import lax
from jax.experimental import pallas as pl
from jax.experimental.pallas import tpu as pltpu
from jax.experimental.pallas import tpu_sc as plsc

BATCH = 16384
N_PROTO = 256
N_WORKERS = 32
ROWS_PER_WORKER = BATCH // N_WORKERS
GROUP = 16
TILE = 32
N_TILES = ROWS_PER_WORKER // TILE
GROUPS_PER_TILE = TILE // GROUP
N_CHUNKS = N_PROTO // 16
BIG = 2**30


def _wtac_body(dist_hbm, labels_hbm, out_hbm, labels_v, tiles, out_v, sems):
    wid = lax.axis_index("c") * 16 + lax.axis_index("s")
    base = wid * ROWS_PER_WORKER

    pltpu.sync_copy(labels_hbm, labels_v)

    col_iota = lax.iota(jnp.int32, 16)

    def fire(t, b):
        pltpu.async_copy(
            dist_hbm.at[pl.ds(base + t * TILE, TILE), :], tiles[b], sems[b]
        )

    def drain(b):
        pltpu.make_async_copy(
            dist_hbm.at[pl.ds(0, TILE), :], tiles[b], sems[b]
        ).wait()

    fire(0, 0)
    fire(1, 1)

    def outer(o, carry):
        for b in range(2):
            t = 2 * o + b
            drain(b)
            tile = tiles[b]

            def group_body(gg, c2):
                r0 = gg * GROUP
                labvec = jnp.zeros((16,), jnp.int32)
                for r in range(GROUP):
                    best = tile[r0 + r, pl.ds(0, 16)]
                    bidx = jnp.zeros((16,), jnp.int32)
                    for j in range(1, N_CHUNKS):
                        v = tile[r0 + r, pl.ds(j * 16, 16)]
                        pred = v < best
                        best = jnp.where(pred, v, best)
                        bidx = jnp.where(
                            pred, jnp.full((16,), j, jnp.int32), bidx
                        )
                    m = jnp.min(best)
                    cand = jnp.where(
                        best == m,
                        bidx * 16 + col_iota,
                        jnp.full((16,), BIG, jnp.int32),
                    )
                    w = jnp.min(cand)
                    labvec = jnp.where(col_iota == r, w, labvec)
                out_v[pl.ds(t * TILE + r0, GROUP)] = plsc.load_gather(
                    labels_v, [labvec]
                )
                return c2

            lax.fori_loop(0, GROUPS_PER_TILE, group_body, 0)

            @pl.when(t + 2 < N_TILES)
            def _():
                fire(t + 2, b)
        return carry

    lax.fori_loop(0, N_TILES // 2, outer, 0)
    pltpu.sync_copy(out_v, out_hbm.at[pl.ds(base, ROWS_PER_WORKER)])


@jax.jit
def _wtac(distances, labels):
    mesh = plsc.VectorSubcoreMesh(core_axis_name="c", subcore_axis_name="s")
    run = pl.kernel(
        _wtac_body,
        out_type=jax.ShapeDtypeStruct((BATCH,), jnp.int32),
        mesh=mesh,
        scratch_types=[
            pltpu.VMEM((N_PROTO,), jnp.int32),
            [pltpu.VMEM((TILE, N_PROTO), jnp.float32)] * 2,
            pltpu.VMEM((ROWS_PER_WORKER,), jnp.int32),
            [pltpu.SemaphoreType.DMA] * 2,
        ],
        compiler_params=pltpu.CompilerParams(
            use_tc_tiling_on_sc=True, needs_layout_passes=False
        ),
        name="wtac_sc",
    )
    return run(distances, labels)


def kernel(distances, prototype_labels):
    labels = prototype_labels.astype(jnp.int32)
    return _wtac(distances, labels)


# speedup vs baseline: 4.3168x; 1.0191x over previous
import jax
import jax.numpy as jnp
from jax import lax
from jax.experimental import pallas as pl
from jax.experimental.pallas import tpu as pltpu
from jax.experimental.pallas import tpu_sc as plsc

BATCH = 16384
N_PROTO = 256
N_WORKERS = 32
ROWS_PER_WORKER = BATCH // N_WORKERS
GROUP = 16
TILE = 32
N_TILES = ROWS_PER_WORKER // TILE
GROUPS_PER_TILE = TILE // GROUP
N_CHUNKS = N_PROTO // 16
BIG = 2**30


def _wtac_body(dist_hbm, labels_hbm, out_hbm, labels_v, tiles, out_v, sems):
    wid = lax.axis_index("c") * 16 + lax.axis_index("s")
    base = wid * ROWS_PER_WORKER

    pltpu.sync_copy(labels_hbm, labels_v)

    col_iota = lax.iota(jnp.int32, 16)

    def fire(t, b):
        pltpu.async_copy(
            dist_hbm.at[pl.ds(base + t * TILE, TILE), :], tiles[b], sems[b]
        )

    def drain(b):
        pltpu.make_async_copy(
            dist_hbm.at[pl.ds(0, TILE), :], tiles[b], sems[b]
        ).wait()

    fire(0, 0)
    fire(1, 1)

    def outer(o, carry):
        for b in range(2):
            t = 2 * o + b
            drain(b)
            tile = tiles[b]

            def group_body(gg, c2):
                r0 = gg * GROUP

                def half_body(h, labvec):
                    rbase = r0 + h * 8
                    for r in range(8):
                        row = rbase + r
                        best0 = tile[row, pl.ds(0, 16)]
                        bidx0 = jnp.zeros((16,), jnp.int32)
                        best1 = tile[row, pl.ds(128, 16)]
                        bidx1 = jnp.full((16,), 8, jnp.int32)
                        for j in range(1, 8):
                            v0 = tile[row, pl.ds(j * 16, 16)]
                            p0 = v0 < best0
                            best0 = jnp.where(p0, v0, best0)
                            bidx0 = jnp.where(
                                p0, jnp.full((16,), j, jnp.int32), bidx0
                            )
                            v1 = tile[row, pl.ds(128 + j * 16, 16)]
                            p1 = v1 < best1
                            best1 = jnp.where(p1, v1, best1)
                            bidx1 = jnp.where(
                                p1, jnp.full((16,), j + 8, jnp.int32), bidx1
                            )
                        pm = best1 < best0
                        best = jnp.where(pm, best1, best0)
                        bidx = jnp.where(pm, bidx1, bidx0)
                        m = jnp.min(best)
                        cand = jnp.where(
                            best == m,
                            bidx * 16 + col_iota,
                            jnp.full((16,), BIG, jnp.int32),
                        )
                        w = jnp.min(cand)
                        labvec = jnp.where(col_iota == (h * 8 + r), w, labvec)
                    return labvec

                labvec = lax.fori_loop(
                    0, 2, half_body, jnp.zeros((16,), jnp.int32)
                )
                out_v[pl.ds(t * TILE + r0, GROUP)] = plsc.load_gather(
                    labels_v, [labvec]
                )
                return c2

            lax.fori_loop(0, GROUPS_PER_TILE, group_body, 0)

            @pl.when(t + 2 < N_TILES)
            def _():
                fire(t + 2, b)
        return carry

    lax.fori_loop(0, N_TILES // 2, outer, 0)
    pltpu.sync_copy(out_v, out_hbm.at[pl.ds(base, ROWS_PER_WORKER)])


@jax.jit
def _wtac(distances, labels):
    mesh = plsc.VectorSubcoreMesh(core_axis_name="c", subcore_axis_name="s")
    run = pl.kernel(
        _wtac_body,
        out_type=jax.ShapeDtypeStruct((BATCH,), jnp.int32),
        mesh=mesh,
        scratch_types=[
            pltpu.VMEM((N_PROTO,), jnp.int32),
            [pltpu.VMEM((TILE, N_PROTO), jnp.float32)] * 2,
            pltpu.VMEM((ROWS_PER_WORKER,), jnp.int32),
            [pltpu.SemaphoreType.DMA] * 2,
        ],
        compiler_params=pltpu.CompilerParams(
            use_tc_tiling_on_sc=True, needs_layout_passes=False
        ),
        name="wtac_sc",
    )
    return run(distances, labels)


def kernel(distances, prototype_labels):
    labels = prototype_labels.astype(jnp.int32)
    return _wtac(distances, labels)


# speedup vs baseline: 4.5703x; 1.0587x over previous
import jax
import jax.numpy as jnp
from jax import lax
from jax.experimental import pallas as pl
from jax.experimental.pallas import tpu as pltpu
from jax.experimental.pallas import tpu_sc as plsc

BATCH = 16384
N_PROTO = 256
N_WORKERS = 32
ROWS_PER_WORKER = BATCH // N_WORKERS
GROUP = 16
TILE = 64
N_TILES = ROWS_PER_WORKER // TILE
GROUPS_PER_TILE = TILE // GROUP
N_CHUNKS = N_PROTO // 16
BIG = 2**30


def _wtac_body(dist_hbm, labels_hbm, out_hbm, labels_v, tiles, out_v, sems):
    wid = lax.axis_index("c") * 16 + lax.axis_index("s")
    base = wid * ROWS_PER_WORKER

    pltpu.sync_copy(labels_hbm, labels_v)

    col_iota = lax.iota(jnp.int32, 16)

    def fire(t, b):
        pltpu.async_copy(
            dist_hbm.at[pl.ds(base + t * TILE, TILE), :], tiles[b], sems[b]
        )

    def drain(b):
        pltpu.make_async_copy(
            dist_hbm.at[pl.ds(0, TILE), :], tiles[b], sems[b]
        ).wait()

    fire(0, 0)
    fire(1, 1)

    def outer(o, carry):
        for b in range(2):
            t = 2 * o + b
            drain(b)
            tile = tiles[b]

            def group_body(gg, c2):
                r0 = gg * GROUP

                def half_body(h, labvec):
                    rbase = r0 + h * 8
                    for r in range(8):
                        row = rbase + r
                        best0 = tile[row, pl.ds(0, 16)]
                        bidx0 = jnp.zeros((16,), jnp.int32)
                        best1 = tile[row, pl.ds(128, 16)]
                        bidx1 = jnp.full((16,), 8, jnp.int32)
                        for j in range(1, 8):
                            v0 = tile[row, pl.ds(j * 16, 16)]
                            p0 = v0 < best0
                            best0 = jnp.where(p0, v0, best0)
                            bidx0 = jnp.where(
                                p0, jnp.full((16,), j, jnp.int32), bidx0
                            )
                            v1 = tile[row, pl.ds(128 + j * 16, 16)]
                            p1 = v1 < best1
                            best1 = jnp.where(p1, v1, best1)
                            bidx1 = jnp.where(
                                p1, jnp.full((16,), j + 8, jnp.int32), bidx1
                            )
                        pm = best1 < best0
                        best = jnp.where(pm, best1, best0)
                        bidx = jnp.where(pm, bidx1, bidx0)
                        m = jnp.min(best)
                        cand = jnp.where(
                            best == m,
                            bidx * 16 + col_iota,
                            jnp.full((16,), BIG, jnp.int32),
                        )
                        w = jnp.min(cand)
                        labvec = jnp.where(col_iota == (h * 8 + r), w, labvec)
                    return labvec

                labvec = lax.fori_loop(
                    0, 2, half_body, jnp.zeros((16,), jnp.int32)
                )
                out_v[pl.ds(t * TILE + r0, GROUP)] = plsc.load_gather(
                    labels_v, [labvec]
                )
                return c2

            lax.fori_loop(0, GROUPS_PER_TILE, group_body, 0)

            @pl.when(t + 2 < N_TILES)
            def _():
                fire(t + 2, b)
        return carry

    lax.fori_loop(0, N_TILES // 2, outer, 0)
    pltpu.sync_copy(out_v, out_hbm.at[pl.ds(base, ROWS_PER_WORKER)])


@jax.jit
def _wtac(distances, labels):
    mesh = plsc.VectorSubcoreMesh(core_axis_name="c", subcore_axis_name="s")
    run = pl.kernel(
        _wtac_body,
        out_type=jax.ShapeDtypeStruct((BATCH,), jnp.int32),
        mesh=mesh,
        scratch_types=[
            pltpu.VMEM((N_PROTO,), jnp.int32),
            [pltpu.VMEM((TILE, N_PROTO), jnp.float32)] * 2,
            pltpu.VMEM((ROWS_PER_WORKER,), jnp.int32),
            [pltpu.SemaphoreType.DMA] * 2,
        ],
        compiler_params=pltpu.CompilerParams(
            use_tc_tiling_on_sc=True, needs_layout_passes=False,
            disable_bounds_checks=True, disable_semaphore_checks=True,
            skip_device_barrier=True
        ),
        name="wtac_sc",
    )
    return run(distances, labels)


def kernel(distances, prototype_labels):
    labels = prototype_labels.astype(jnp.int32)
    return _wtac(distances, labels)
